# Initial kernel scaffold; baseline (speedup 1.0000x reference)
import functools

import jax
import jax.numpy as jnp
from jax import lax
from jax.experimental import pallas as pl
from jax.experimental.pallas import tpu as pltpu
from jax.experimental.pallas import tpu_sc as plsc

N = 10000
E = 320000
D_IN = 128
H = 32
G = 64
D_OUT = 128

NC = 2
NS = 16
NW = NC * NS

CHUNK = 80
OPS_PER_TILE = E // (NW * CHUNK)
ROWS_PER_TILE = N // NS


_sc_mesh = plsc.VectorSubcoreMesh(core_axis_name="c", subcore_axis_name="s")


@functools.partial(
    pl.kernel,
    out_type=jax.ShapeDtypeStruct((NC * N, H), jnp.float32),
    mesh=_sc_mesh,
    scratch_types=[
        pltpu.VMEM((CHUNK,), jnp.int32),
        pltpu.VMEM((CHUNK,), jnp.int32),
        pltpu.VMEM((CHUNK, H), jnp.float32),
        pltpu.VMEM((ROWS_PER_TILE, H), jnp.float32),
        pltpu.VMEM_SHARED((N, H), jnp.float32),
        pltpu.SemaphoreType.DMA,
    ],
)
def _sc_segment_sum(z_hbm, src_hbm, dst_hbm, out_hbm,
                    sidx, didx, rows, ztile, acc, sem):
    c = lax.axis_index("c")
    s = lax.axis_index("s")
    wid = s * NC + c

    ztile[...] = jnp.zeros_like(ztile)
    pltpu.sync_copy(ztile, acc.at[pl.ds(s * ROWS_PER_TILE, ROWS_PER_TILE)])
    plsc.subcore_barrier()

    def body(k, carry):
        base = wid * (OPS_PER_TILE * CHUNK) + k * CHUNK
        pltpu.sync_copy(src_hbm.at[pl.ds(base, CHUNK)], sidx)
        pltpu.async_copy(z_hbm.at[sidx], rows, sem).wait()
        pltpu.sync_copy(dst_hbm.at[pl.ds(base, CHUNK)], didx)
        pltpu.sync_copy(rows, acc.at[didx], add=True)
        return carry

    lax.fori_loop(0, OPS_PER_TILE, body, 0)
    plsc.subcore_barrier()

    r0 = s * ROWS_PER_TILE
    pltpu.sync_copy(acc.at[pl.ds(r0, ROWS_PER_TILE)],
                    out_hbm.at[pl.ds(c * N + r0, ROWS_PER_TILE)])


def _proj_body(x_ref, w_ref, o_ref):
    o_ref[...] = jnp.dot(x_ref[...], w_ref[...],
                         preferred_element_type=jnp.float32)


def _proj(x, w):
    return pl.pallas_call(
        _proj_body,
        out_shape=jax.ShapeDtypeStruct((x.shape[0], w.shape[1]), jnp.float32),
    )(x, w)


def _bn_tail(pre, gamma_ref, beta_ref, w2_ref, b2_ref):
    mean = jnp.mean(pre, axis=0, keepdims=True)
    var = jnp.mean((pre - mean) ** 2, axis=0, keepdims=True)
    hn = (pre - mean) * lax.rsqrt(var + 1e-5) * gamma_ref[...] + beta_ref[...]
    hn = jnp.maximum(hn, 0.0)
    h2 = jnp.dot(hn, w2_ref[...], preferred_element_type=jnp.float32) + b2_ref[...]
    return jnp.maximum(h2, 0.0)


def _mid_body(z_ref, part_ref, b1_ref, gamma_ref, beta_ref, w2_ref, b2_ref,
              wn_ref, o_ref):
    pre = z_ref[...] + part_ref[0] + part_ref[1] + b1_ref[...]
    h2 = _bn_tail(pre, gamma_ref, beta_ref, w2_ref, b2_ref)
    o_ref[...] = jnp.dot(h2, wn_ref[...], preferred_element_type=jnp.float32)


def _mid(z, part, p, w_next):
    part3 = part.reshape(NC, N, H)
    return pl.pallas_call(
        _mid_body,
        out_shape=jax.ShapeDtypeStruct((N, w_next.shape[1]), jnp.float32),
    )(z, part3, p["b1"].reshape(1, H), p["gamma"].reshape(1, H),
      p["beta"].reshape(1, H), p["W2"], p["b2"].reshape(1, H), w_next)


def _last_body(z_ref, part_ref, b1_ref, gamma_ref, beta_ref, w2_ref, b2_ref,
               hw1_ref, batch_ref, hb1_ref, hw2_ref, hb2_ref, o_ref):
    pre = z_ref[...] + part_ref[0] + part_ref[1] + b1_ref[...]
    h2 = _bn_tail(pre, gamma_ref, beta_ref, w2_ref, b2_ref)
    hz = jnp.dot(h2, hw1_ref[...], preferred_element_type=jnp.float32)
    gids = lax.broadcasted_iota(jnp.int32, (G, N), 0)
    onehot = jnp.where(gids == batch_ref[...], 1.0, 0.0)
    pooled = jnp.dot(onehot, hz, preferred_element_type=jnp.float32)
    hh = jnp.maximum(pooled + hb1_ref[...], 0.0)
    o_ref[...] = jnp.dot(hh, hw2_ref[...],
                         preferred_element_type=jnp.float32) + hb2_ref[...]


def _last(z, part, p, head, batch):
    part3 = part.reshape(NC, N, H)
    return pl.pallas_call(
        _last_body,
        out_shape=jax.ShapeDtypeStruct((G, D_OUT), jnp.float32),
    )(z, part3, p["b1"].reshape(1, H), p["gamma"].reshape(1, H),
      p["beta"].reshape(1, H), p["W2"], p["b2"].reshape(1, H),
      head["W1"], batch.reshape(1, N), head["b1"].reshape(1, H),
      head["W2"], head["b2"].reshape(1, D_OUT))


def kernel(x, edge_index, batch, params):
    src = edge_index[0]
    dst = edge_index[1]
    convs = params["convs"]
    z = _proj(x, convs[0]["W1"])
    for l in range(5):
        part = _sc_segment_sum(z, src, dst)
        if l < 4:
            z = _mid(z, part, convs[l], convs[l + 1]["W1"])
        else:
            out = _last(z, part, convs[4], params["head"], batch)
    return out


# speedup vs baseline: 5.6363x; 5.6363x over previous
import functools

import jax
import jax.numpy as jnp
from jax import lax
from jax.experimental import pallas as pl
from jax.experimental.pallas import tpu as pltpu
from jax.experimental.pallas import tpu_sc as plsc

N = 10000
E = 320000
D_IN = 128
H = 32
G = 64
D_OUT = 128

NC = 2
NS = 16
NW = NC * NS

CHUNK = 80
OPS_PER_TILE = E // (NW * CHUNK)
NP = 10240
ROWS_PER_TILE = NP // NS


_sc_mesh = plsc.VectorSubcoreMesh(core_axis_name="c", subcore_axis_name="s")


@functools.partial(
    pl.kernel,
    out_type=jax.ShapeDtypeStruct((NC * NP, H), jnp.float32),
    mesh=_sc_mesh,
    scratch_types=[
        pltpu.VMEM((CHUNK,), jnp.int32),
        pltpu.VMEM((CHUNK,), jnp.int32),
        pltpu.VMEM((CHUNK, H), jnp.float32),
        pltpu.VMEM_SHARED((NP, H), jnp.float32),
        pltpu.SemaphoreType.DMA,
    ],
    compiler_params=pltpu.CompilerParams(use_tc_tiling_on_sc=False),
)
def _sc_segment_sum(z_hbm, src_hbm, dst_hbm, zeros_hbm, out_hbm,
                    sidx, didx, rows, acc, sem):
    c = lax.axis_index("c")
    s = lax.axis_index("s")
    wid = s * NC + c

    r0 = s * ROWS_PER_TILE
    pltpu.sync_copy(zeros_hbm.at[pl.ds(r0, ROWS_PER_TILE)],
                    acc.at[pl.ds(r0, ROWS_PER_TILE)])
    plsc.subcore_barrier()

    def body(k, carry):
        base = wid * (OPS_PER_TILE * CHUNK) + k * CHUNK
        pltpu.sync_copy(src_hbm.at[pl.ds(base, CHUNK)], sidx)
        pltpu.async_copy(z_hbm.at[sidx], rows, sem).wait()
        pltpu.sync_copy(dst_hbm.at[pl.ds(base, CHUNK)], didx)
        pltpu.sync_copy(rows, acc.at[didx], add=True)
        return carry

    lax.fori_loop(0, OPS_PER_TILE, body, 0)
    plsc.subcore_barrier()

    pltpu.sync_copy(acc.at[pl.ds(r0, ROWS_PER_TILE)],
                    out_hbm.at[pl.ds(c * NP + r0, ROWS_PER_TILE)])


def _proj_body(x_ref, w_ref, o_ref):
    o_ref[...] = jnp.dot(x_ref[...], w_ref[...],
                         preferred_element_type=jnp.float32)


def _proj(x, w):
    return pl.pallas_call(
        _proj_body,
        out_shape=jax.ShapeDtypeStruct((x.shape[0], w.shape[1]), jnp.float32),
    )(x, w)


def _bn_tail(pre, gamma_ref, beta_ref, w2_ref, b2_ref):
    mean = jnp.mean(pre, axis=0, keepdims=True)
    var = jnp.mean((pre - mean) ** 2, axis=0, keepdims=True)
    hn = (pre - mean) * lax.rsqrt(var + 1e-5) * gamma_ref[...] + beta_ref[...]
    hn = jnp.maximum(hn, 0.0)
    h2 = jnp.dot(hn, w2_ref[...], preferred_element_type=jnp.float32) + b2_ref[...]
    return jnp.maximum(h2, 0.0)


def _mid_body(z_ref, part_ref, b1_ref, gamma_ref, beta_ref, w2_ref, b2_ref,
              wn_ref, o_ref):
    pre = z_ref[...] + part_ref[0, :N, :] + part_ref[1, :N, :] + b1_ref[...]
    h2 = _bn_tail(pre, gamma_ref, beta_ref, w2_ref, b2_ref)
    o_ref[...] = jnp.dot(h2, wn_ref[...], preferred_element_type=jnp.float32)


def _mid(z, part, p, w_next):
    part3 = part.reshape(NC, NP, H)
    return pl.pallas_call(
        _mid_body,
        out_shape=jax.ShapeDtypeStruct((N, w_next.shape[1]), jnp.float32),
    )(z, part3, p["b1"].reshape(1, H), p["gamma"].reshape(1, H),
      p["beta"].reshape(1, H), p["W2"], p["b2"].reshape(1, H), w_next)


def _last_body(z_ref, part_ref, b1_ref, gamma_ref, beta_ref, w2_ref, b2_ref,
               hw1_ref, batch_ref, hb1_ref, hw2_ref, hb2_ref, o_ref):
    pre = z_ref[...] + part_ref[0, :N, :] + part_ref[1, :N, :] + b1_ref[...]
    h2 = _bn_tail(pre, gamma_ref, beta_ref, w2_ref, b2_ref)
    hz = jnp.dot(h2, hw1_ref[...], preferred_element_type=jnp.float32)
    gids = lax.broadcasted_iota(jnp.int32, (G, N), 0)
    onehot = jnp.where(gids == batch_ref[...], 1.0, 0.0)
    pooled = jnp.dot(onehot, hz, preferred_element_type=jnp.float32)
    hh = jnp.maximum(pooled + hb1_ref[...], 0.0)
    o_ref[...] = jnp.dot(hh, hw2_ref[...],
                         preferred_element_type=jnp.float32) + hb2_ref[...]


def _last(z, part, p, head, batch):
    part3 = part.reshape(NC, NP, H)
    return pl.pallas_call(
        _last_body,
        out_shape=jax.ShapeDtypeStruct((G, D_OUT), jnp.float32),
    )(z, part3, p["b1"].reshape(1, H), p["gamma"].reshape(1, H),
      p["beta"].reshape(1, H), p["W2"], p["b2"].reshape(1, H),
      head["W1"], batch.reshape(1, N), head["b1"].reshape(1, H),
      head["W2"], head["b2"].reshape(1, D_OUT))


def kernel(x, edge_index, batch, params):
    src = edge_index[0]
    dst = edge_index[1]
    convs = params["convs"]
    zeros = jnp.zeros((NP, H), jnp.float32)
    z = _proj(x, convs[0]["W1"])
    for l in range(5):
        part = _sc_segment_sum(z, src, dst, zeros)
        if l < 4:
            z = _mid(z, part, convs[l], convs[l + 1]["W1"])
        else:
            out = _last(z, part, convs[4], params["head"], batch)
    return out


# speedup vs baseline: 15.7274x; 2.7904x over previous
import functools

import jax
import jax.numpy as jnp
from jax import lax
from jax.experimental import pallas as pl
from jax.experimental.pallas import tpu as pltpu
from jax.experimental.pallas import tpu_sc as plsc

N = 10000
E = 320000
D_IN = 128
H = 32
G = 64
D_OUT = 128

NC = 2
NS = 16
NW = NC * NS

CHUNK = 80
OPS_PER_TILE = E // (NW * CHUNK)
NP = 10240
ROWS_PER_TILE = NP // NS


_sc_mesh = plsc.VectorSubcoreMesh(core_axis_name="c", subcore_axis_name="s")


@functools.partial(
    pl.kernel,
    out_type=jax.ShapeDtypeStruct((NC * NP, H), jnp.float32),
    mesh=_sc_mesh,
    scratch_types=[
        pltpu.VMEM((OPS_PER_TILE, CHUNK), jnp.int32),
        pltpu.VMEM((OPS_PER_TILE, CHUNK), jnp.int32),
        pltpu.VMEM((CHUNK, H), jnp.float32),
        pltpu.VMEM_SHARED((NP, H), jnp.float32),
        pltpu.VMEM_SHARED((NP, H), jnp.float32),
        pltpu.SemaphoreType.DMA,
    ],
    compiler_params=pltpu.CompilerParams(use_tc_tiling_on_sc=False),
)
def _sc_segment_sum(z_hbm, src_hbm, dst_hbm, zeros_hbm, out_hbm,
                    sidx, didx, rows, zsp, acc, sem):
    c = lax.axis_index("c")
    s = lax.axis_index("s")
    wid = s * NC + c
    r0 = s * ROWS_PER_TILE

    pltpu.sync_copy(src_hbm.at[pl.ds(wid * OPS_PER_TILE, OPS_PER_TILE)], sidx)
    pltpu.sync_copy(dst_hbm.at[pl.ds(wid * OPS_PER_TILE, OPS_PER_TILE)], didx)
    pltpu.sync_copy(z_hbm.at[pl.ds(r0, ROWS_PER_TILE)],
                    zsp.at[pl.ds(r0, ROWS_PER_TILE)])
    pltpu.sync_copy(zeros_hbm.at[pl.ds(r0, ROWS_PER_TILE)],
                    acc.at[pl.ds(r0, ROWS_PER_TILE)])
    plsc.subcore_barrier()

    def body(k, carry):
        pltpu.async_copy(zsp.at[sidx.at[k]], rows, sem).wait()
        pltpu.sync_copy(rows, acc.at[didx.at[k]], add=True)
        return carry

    lax.fori_loop(0, OPS_PER_TILE, body, 0)
    plsc.subcore_barrier()

    pltpu.sync_copy(acc.at[pl.ds(r0, ROWS_PER_TILE)],
                    out_hbm.at[pl.ds(c * NP + r0, ROWS_PER_TILE)])


def _proj_body(x_ref, w_ref, o_ref):
    o_ref[:N, :] = jnp.dot(x_ref[...], w_ref[...],
                           preferred_element_type=jnp.float32)
    o_ref[N:, :] = jnp.zeros((NP - N, H), jnp.float32)


def _proj(x, w):
    return pl.pallas_call(
        _proj_body,
        out_shape=jax.ShapeDtypeStruct((NP, w.shape[1]), jnp.float32),
    )(x, w)


def _bn_tail(pre, gamma_ref, beta_ref, w2_ref, b2_ref):
    mean = jnp.mean(pre, axis=0, keepdims=True)
    var = jnp.mean((pre - mean) ** 2, axis=0, keepdims=True)
    hn = (pre - mean) * lax.rsqrt(var + 1e-5) * gamma_ref[...] + beta_ref[...]
    hn = jnp.maximum(hn, 0.0)
    h2 = jnp.dot(hn, w2_ref[...], preferred_element_type=jnp.float32) + b2_ref[...]
    return jnp.maximum(h2, 0.0)


def _mid_body(z_ref, part_ref, b1_ref, gamma_ref, beta_ref, w2_ref, b2_ref,
              wn_ref, o_ref):
    pre = (z_ref[:N, :] + part_ref[0, :N, :] + part_ref[1, :N, :]
           + b1_ref[...])
    h2 = _bn_tail(pre, gamma_ref, beta_ref, w2_ref, b2_ref)
    o_ref[:N, :] = jnp.dot(h2, wn_ref[...], preferred_element_type=jnp.float32)
    o_ref[N:, :] = jnp.zeros((NP - N, H), jnp.float32)


def _mid(z, part, p, w_next):
    part3 = part.reshape(NC, NP, H)
    return pl.pallas_call(
        _mid_body,
        out_shape=jax.ShapeDtypeStruct((NP, w_next.shape[1]), jnp.float32),
    )(z, part3, p["b1"].reshape(1, H), p["gamma"].reshape(1, H),
      p["beta"].reshape(1, H), p["W2"], p["b2"].reshape(1, H), w_next)


def _last_body(z_ref, part_ref, b1_ref, gamma_ref, beta_ref, w2_ref, b2_ref,
               hw1_ref, batch_ref, hb1_ref, hw2_ref, hb2_ref, o_ref):
    pre = (z_ref[:N, :] + part_ref[0, :N, :] + part_ref[1, :N, :]
           + b1_ref[...])
    h2 = _bn_tail(pre, gamma_ref, beta_ref, w2_ref, b2_ref)
    hz = jnp.dot(h2, hw1_ref[...], preferred_element_type=jnp.float32)
    gids = lax.broadcasted_iota(jnp.int32, (G, N), 0)
    onehot = jnp.where(gids == batch_ref[...], 1.0, 0.0)
    pooled = jnp.dot(onehot, hz, preferred_element_type=jnp.float32)
    hh = jnp.maximum(pooled + hb1_ref[...], 0.0)
    o_ref[...] = jnp.dot(hh, hw2_ref[...],
                         preferred_element_type=jnp.float32) + hb2_ref[...]


def _last(z, part, p, head, batch):
    part3 = part.reshape(NC, NP, H)
    return pl.pallas_call(
        _last_body,
        out_shape=jax.ShapeDtypeStruct((G, D_OUT), jnp.float32),
    )(z, part3, p["b1"].reshape(1, H), p["gamma"].reshape(1, H),
      p["beta"].reshape(1, H), p["W2"], p["b2"].reshape(1, H),
      head["W1"], batch.reshape(1, N), head["b1"].reshape(1, H),
      head["W2"], head["b2"].reshape(1, D_OUT))


def kernel(x, edge_index, batch, params):
    src = edge_index[0].reshape(E // CHUNK, CHUNK)
    dst = edge_index[1].reshape(E // CHUNK, CHUNK)
    convs = params["convs"]
    zeros = jnp.zeros((NP, H), jnp.float32)
    z = _proj(x, convs[0]["W1"])
    for l in range(5):
        part = _sc_segment_sum(z, src, dst, zeros)
        if l < 4:
            z = _mid(z, part, convs[l], convs[l + 1]["W1"])
        else:
            out = _last(z, part, convs[4], params["head"], batch)
    return out


# speedup vs baseline: 19.2360x; 1.2231x over previous
import functools

import jax
import jax.numpy as jnp
from jax import lax
from jax.experimental import pallas as pl
from jax.experimental.pallas import tpu as pltpu
from jax.experimental.pallas import tpu_sc as plsc

N = 10000
E = 320000
D_IN = 128
H = 32
G = 64
D_OUT = 128

NC = 2
NS = 16
NW = NC * NS

CHUNK = 80
OPS_PER_TILE = E // (NW * CHUNK)
NP = 10240
ROWS_PER_TILE = NP // NS


_sc_mesh = plsc.VectorSubcoreMesh(core_axis_name="c", subcore_axis_name="s")


@functools.partial(
    pl.kernel,
    out_type=jax.ShapeDtypeStruct((NC * NP, H), jnp.float32),
    mesh=_sc_mesh,
    scratch_types=[
        pltpu.VMEM((OPS_PER_TILE, CHUNK), jnp.int32),
        pltpu.VMEM((OPS_PER_TILE, CHUNK), jnp.int32),
        pltpu.VMEM((CHUNK, H), jnp.float32),
        pltpu.VMEM((CHUNK, H), jnp.float32),
        pltpu.VMEM_SHARED((NP, H), jnp.float32),
        pltpu.VMEM_SHARED((NP, H), jnp.float32),
        pltpu.SemaphoreType.DMA,
        pltpu.SemaphoreType.DMA,
    ],
    compiler_params=pltpu.CompilerParams(use_tc_tiling_on_sc=False),
)
def _sc_segment_sum(z_hbm, src_hbm, dst_hbm, zeros_hbm, out_hbm,
                    sidx, didx, rows0, rows1, zsp, acc, sem0, sem1):
    c = lax.axis_index("c")
    s = lax.axis_index("s")
    wid = s * NC + c
    r0 = s * ROWS_PER_TILE

    pltpu.sync_copy(src_hbm.at[pl.ds(wid * OPS_PER_TILE, OPS_PER_TILE)], sidx)
    pltpu.sync_copy(dst_hbm.at[pl.ds(wid * OPS_PER_TILE, OPS_PER_TILE)], didx)
    pltpu.sync_copy(z_hbm.at[pl.ds(r0, ROWS_PER_TILE)],
                    zsp.at[pl.ds(r0, ROWS_PER_TILE)])
    pltpu.sync_copy(zeros_hbm.at[pl.ds(r0, ROWS_PER_TILE)],
                    acc.at[pl.ds(r0, ROWS_PER_TILE)])
    plsc.subcore_barrier()

    def _drain(buf, sem):
        pltpu.make_async_copy(z_hbm.at[pl.ds(0, CHUNK)], buf, sem).wait()

    def body(g, carry):
        k = 2 * g
        pltpu.async_copy(zsp.at[sidx.at[k + 1]], rows1, sem1)
        _drain(rows0, sem0)
        pltpu.sync_copy(rows0, acc.at[didx.at[k]], add=True)
        pltpu.async_copy(zsp.at[sidx.at[k + 2]], rows0, sem0)
        _drain(rows1, sem1)
        pltpu.sync_copy(rows1, acc.at[didx.at[k + 1]], add=True)
        return carry

    pltpu.async_copy(zsp.at[sidx.at[0]], rows0, sem0)
    lax.fori_loop(0, (OPS_PER_TILE - 1) // 2, body, 0)
    _drain(rows0, sem0)
    pltpu.sync_copy(rows0, acc.at[didx.at[OPS_PER_TILE - 1]], add=True)
    plsc.subcore_barrier()

    pltpu.sync_copy(acc.at[pl.ds(r0, ROWS_PER_TILE)],
                    out_hbm.at[pl.ds(c * NP + r0, ROWS_PER_TILE)])


def _proj_body(x_ref, w_ref, o_ref):
    o_ref[:N, :] = jnp.dot(x_ref[...], w_ref[...],
                           preferred_element_type=jnp.float32)
    o_ref[N:, :] = jnp.zeros((NP - N, H), jnp.float32)


def _proj(x, w):
    return pl.pallas_call(
        _proj_body,
        out_shape=jax.ShapeDtypeStruct((NP, w.shape[1]), jnp.float32),
    )(x, w)


def _bn_tail(pre, gamma_ref, beta_ref, w2_ref, b2_ref):
    mean = jnp.mean(pre, axis=0, keepdims=True)
    var = jnp.mean((pre - mean) ** 2, axis=0, keepdims=True)
    hn = (pre - mean) * lax.rsqrt(var + 1e-5) * gamma_ref[...] + beta_ref[...]
    hn = jnp.maximum(hn, 0.0)
    h2 = jnp.dot(hn, w2_ref[...], preferred_element_type=jnp.float32) + b2_ref[...]
    return jnp.maximum(h2, 0.0)


def _mid_body(z_ref, part_ref, b1_ref, gamma_ref, beta_ref, w2_ref, b2_ref,
              wn_ref, o_ref):
    pre = (z_ref[:N, :] + part_ref[0, :N, :] + part_ref[1, :N, :]
           + b1_ref[...])
    h2 = _bn_tail(pre, gamma_ref, beta_ref, w2_ref, b2_ref)
    o_ref[:N, :] = jnp.dot(h2, wn_ref[...], preferred_element_type=jnp.float32)
    o_ref[N:, :] = jnp.zeros((NP - N, H), jnp.float32)


def _mid(z, part, p, w_next):
    part3 = part.reshape(NC, NP, H)
    return pl.pallas_call(
        _mid_body,
        out_shape=jax.ShapeDtypeStruct((NP, w_next.shape[1]), jnp.float32),
    )(z, part3, p["b1"].reshape(1, H), p["gamma"].reshape(1, H),
      p["beta"].reshape(1, H), p["W2"], p["b2"].reshape(1, H), w_next)


def _last_body(z_ref, part_ref, b1_ref, gamma_ref, beta_ref, w2_ref, b2_ref,
               hw1_ref, batch_ref, hb1_ref, hw2_ref, hb2_ref, o_ref):
    pre = (z_ref[:N, :] + part_ref[0, :N, :] + part_ref[1, :N, :]
           + b1_ref[...])
    h2 = _bn_tail(pre, gamma_ref, beta_ref, w2_ref, b2_ref)
    hz = jnp.dot(h2, hw1_ref[...], preferred_element_type=jnp.float32)
    gids = lax.broadcasted_iota(jnp.int32, (G, N), 0)
    onehot = jnp.where(gids == batch_ref[...], 1.0, 0.0)
    pooled = jnp.dot(onehot, hz, preferred_element_type=jnp.float32)
    hh = jnp.maximum(pooled + hb1_ref[...], 0.0)
    o_ref[...] = jnp.dot(hh, hw2_ref[...],
                         preferred_element_type=jnp.float32) + hb2_ref[...]


def _last(z, part, p, head, batch):
    part3 = part.reshape(NC, NP, H)
    return pl.pallas_call(
        _last_body,
        out_shape=jax.ShapeDtypeStruct((G, D_OUT), jnp.float32),
    )(z, part3, p["b1"].reshape(1, H), p["gamma"].reshape(1, H),
      p["beta"].reshape(1, H), p["W2"], p["b2"].reshape(1, H),
      head["W1"], batch.reshape(1, N), head["b1"].reshape(1, H),
      head["W2"], head["b2"].reshape(1, D_OUT))


def kernel(x, edge_index, batch, params):
    src = edge_index[0].reshape(E // CHUNK, CHUNK)
    dst = edge_index[1].reshape(E // CHUNK, CHUNK)
    convs = params["convs"]
    zeros = jnp.zeros((NP, H), jnp.float32)
    z = _proj(x, convs[0]["W1"])
    for l in range(5):
        part = _sc_segment_sum(z, src, dst, zeros)
        if l < 4:
            z = _mid(z, part, convs[l], convs[l + 1]["W1"])
        else:
            out = _last(z, part, convs[4], params["head"], batch)
    return out


# speedup vs baseline: 24.4730x; 1.2723x over previous
import functools

import jax
import jax.numpy as jnp
from jax import lax
from jax.experimental import pallas as pl
from jax.experimental.pallas import tpu as pltpu
from jax.experimental.pallas import tpu_sc as plsc

N = 10000
E = 320000
D_IN = 128
H = 32
G = 64
D_OUT = 128

NC = 2
NS = 16
NW = NC * NS

CHUNK = 128
EP = 327680
OPS_PER_TILE = EP // (NW * CHUNK)
NP = 10240
ROWS_PER_TILE = NP // NS


_sc_mesh = plsc.VectorSubcoreMesh(core_axis_name="c", subcore_axis_name="s")


@functools.partial(
    pl.kernel,
    out_type=jax.ShapeDtypeStruct((NC * NP, H), jnp.float32),
    mesh=_sc_mesh,
    scratch_types=[
        pltpu.VMEM((OPS_PER_TILE, CHUNK), jnp.int32),
        pltpu.VMEM((OPS_PER_TILE, CHUNK), jnp.int32),
        pltpu.VMEM((CHUNK, H), jnp.float32),
        pltpu.VMEM((CHUNK, H), jnp.float32),
        pltpu.VMEM_SHARED((NP, H), jnp.float32),
        pltpu.VMEM_SHARED((NP, H), jnp.float32),
        pltpu.SemaphoreType.DMA,
        pltpu.SemaphoreType.DMA,
    ],
    compiler_params=pltpu.CompilerParams(use_tc_tiling_on_sc=False),
)
def _sc_segment_sum(z_hbm, src_hbm, dst_hbm, zeros_hbm, out_hbm,
                    sidx, didx, rows0, rows1, zsp, acc, sem0, sem1):
    c = lax.axis_index("c")
    s = lax.axis_index("s")
    wid = s * NC + c
    r0 = s * ROWS_PER_TILE

    pltpu.sync_copy(src_hbm.at[pl.ds(wid * OPS_PER_TILE, OPS_PER_TILE)], sidx)
    pltpu.sync_copy(dst_hbm.at[pl.ds(wid * OPS_PER_TILE, OPS_PER_TILE)], didx)
    pltpu.sync_copy(z_hbm.at[pl.ds(r0, ROWS_PER_TILE)],
                    zsp.at[pl.ds(r0, ROWS_PER_TILE)])
    pltpu.sync_copy(zeros_hbm.at[pl.ds(r0, ROWS_PER_TILE)],
                    acc.at[pl.ds(r0, ROWS_PER_TILE)])
    plsc.subcore_barrier()

    def _drain(buf, sem):
        pltpu.make_async_copy(z_hbm.at[pl.ds(0, CHUNK)], buf, sem).wait()

    def body(g, carry):
        k = 2 * g
        pltpu.async_copy(zsp.at[sidx.at[k + 1]], rows1, sem1)
        _drain(rows0, sem0)
        pltpu.sync_copy(rows0, acc.at[didx.at[k]], add=True)
        pltpu.async_copy(zsp.at[sidx.at[k + 2]], rows0, sem0)
        _drain(rows1, sem1)
        pltpu.sync_copy(rows1, acc.at[didx.at[k + 1]], add=True)
        return carry

    pltpu.async_copy(zsp.at[sidx.at[0]], rows0, sem0)
    lax.fori_loop(0, (OPS_PER_TILE - 2) // 2, body, 0)
    pltpu.async_copy(zsp.at[sidx.at[OPS_PER_TILE - 1]], rows1, sem1)
    _drain(rows0, sem0)
    pltpu.sync_copy(rows0, acc.at[didx.at[OPS_PER_TILE - 2]], add=True)
    _drain(rows1, sem1)
    pltpu.sync_copy(rows1, acc.at[didx.at[OPS_PER_TILE - 1]], add=True)
    plsc.subcore_barrier()

    pltpu.sync_copy(acc.at[pl.ds(r0, ROWS_PER_TILE)],
                    out_hbm.at[pl.ds(c * NP + r0, ROWS_PER_TILE)])


NPR = NP // 4
NRR = N // 4
PW = 4 * H


def _fold4(v):
    return v[:, 0:H] + v[:, H:2 * H] + v[:, 2 * H:3 * H] + v[:, 3 * H:4 * H]


def _tile4(v):
    return jnp.concatenate([v, v, v, v], axis=1)


def _proj_body(x_ref, w_ref, o_ref):
    o_ref[...] = jnp.dot(x_ref[...], w_ref[...],
                         preferred_element_type=jnp.float32)


def _proj(x4, w1bd):
    return pl.pallas_call(
        _proj_body,
        out_shape=jax.ShapeDtypeStruct((NPR, PW), jnp.float32),
    )(x4, w1bd)


def _bn_tail(pre, gamma_ref, beta_ref, w2bd_ref, b2_ref):
    mean = _tile4(_fold4(jnp.sum(pre, axis=0, keepdims=True)) / N)
    d = pre - mean
    var = _tile4(_fold4(jnp.sum(d * d, axis=0, keepdims=True)) / N)
    hn = d * lax.rsqrt(var + 1e-5) * gamma_ref[...] + beta_ref[...]
    hn = jnp.maximum(hn, 0.0)
    h2 = jnp.dot(hn, w2bd_ref[...],
                 preferred_element_type=jnp.float32) + b2_ref[...]
    return jnp.maximum(h2, 0.0)


def _mid_body(z_ref, part_ref, b1_ref, gamma_ref, beta_ref, w2bd_ref, b2_ref,
              wnbd_ref, o_ref):
    pre = (z_ref[:NRR, :] + part_ref[:NRR, :]
           + part_ref[NPR:NPR + NRR, :] + b1_ref[...])
    h2 = _bn_tail(pre, gamma_ref, beta_ref, w2bd_ref, b2_ref)
    o_ref[:NRR, :] = jnp.dot(h2, wnbd_ref[...],
                             preferred_element_type=jnp.float32)
    o_ref[NRR:, :] = jnp.zeros((NPR - NRR, PW), jnp.float32)


def _mid(z, partp, pp):
    return pl.pallas_call(
        _mid_body,
        out_shape=jax.ShapeDtypeStruct((NPR, PW), jnp.float32),
    )(z, partp, pp["b1"], pp["gamma"], pp["beta"], pp["W2bd"], pp["b2"],
      pp["Wnbd"])


def _last_body(z_ref, part_ref, b1_ref, gamma_ref, beta_ref, w2bd_ref, b2_ref,
               hw1bd_ref, batch_ref, hb1_ref, hw2_ref, hb2_ref, o_ref):
    pre = (z_ref[:NRR, :] + part_ref[:NRR, :]
           + part_ref[NPR:NPR + NRR, :] + b1_ref[...])
    h2 = _bn_tail(pre, gamma_ref, beta_ref, w2bd_ref, b2_ref)
    hz = jnp.dot(h2, hw1bd_ref[...], preferred_element_type=jnp.float32)
    gids = lax.broadcasted_iota(jnp.int32, (G, NRR), 0)
    pooled = jnp.zeros((G, H), jnp.float32)
    for j in range(4):
        onehot = jnp.where(gids == batch_ref[j:j + 1, :NRR], 1.0, 0.0)
        pooled = pooled + jnp.dot(onehot, hz[:, j * H:(j + 1) * H],
                                  preferred_element_type=jnp.float32)
    hh = jnp.maximum(pooled + hb1_ref[...], 0.0)
    o_ref[...] = jnp.dot(hh, hw2_ref[...],
                         preferred_element_type=jnp.float32) + hb2_ref[...]


def _last(z, partp, pp, head, batchp):
    return pl.pallas_call(
        _last_body,
        out_shape=jax.ShapeDtypeStruct((G, D_OUT), jnp.float32),
    )(z, partp, pp["b1"], pp["gamma"], pp["beta"], pp["W2bd"], pp["b2"],
      jnp.kron(jnp.eye(4, dtype=jnp.float32), head["W1"]), batchp,
      head["b1"].reshape(1, H), head["W2"], head["b2"].reshape(1, D_OUT))


def _pack_params(p, w_next):
    eye4 = jnp.eye(4, dtype=jnp.float32)
    return {
        "b1": _tile4(p["b1"].reshape(1, H)),
        "gamma": _tile4(p["gamma"].reshape(1, H)),
        "beta": _tile4(p["beta"].reshape(1, H)),
        "W2bd": jnp.kron(eye4, p["W2"]),
        "b2": _tile4(p["b2"].reshape(1, H)),
        "Wnbd": None if w_next is None else jnp.kron(eye4, w_next),
    }


def kernel(x, edge_index, batch, params):
    epad = jnp.full((EP - E,), N, jnp.int32)
    src = jnp.concatenate([edge_index[0], epad]).reshape(EP // CHUNK, CHUNK)
    dst = jnp.concatenate([edge_index[1], epad]).reshape(EP // CHUNK, CHUNK)
    convs = params["convs"]
    zeros = jnp.zeros((NP, H), jnp.float32)
    x4 = jnp.pad(x, ((0, NP - N), (0, 0))).reshape(NPR, 4 * D_IN)
    w1bd0 = jnp.kron(jnp.eye(4, dtype=jnp.float32), convs[0]["W1"])
    batchp = jnp.concatenate(
        [batch, jnp.full((NP - N,), G, jnp.int32)]).reshape(NPR, 4).T
    z = _proj(x4, w1bd0)
    for l in range(5):
        part = _sc_segment_sum(z.reshape(NP, H), src, dst, zeros)
        partp = part.reshape(2 * NPR, PW)
        if l < 4:
            z = _mid(z, partp, _pack_params(convs[l], convs[l + 1]["W1"]))
        else:
            out = _last(z, partp, _pack_params(convs[4], None),
                        params["head"], batchp)
    return out


# speedup vs baseline: 27.4152x; 1.1202x over previous
import functools

import jax
import jax.numpy as jnp
from jax import lax
from jax.experimental import pallas as pl
from jax.experimental.pallas import tpu as pltpu
from jax.experimental.pallas import tpu_sc as plsc

N = 10000
E = 320000
D_IN = 128
H = 32
G = 64
D_OUT = 128

NC = 2
NS = 16
NW = NC * NS

CHUNK = 128
EP = 327680
OPS_PER_TILE = EP // (NW * CHUNK)
NP = 10240
ROWS_PER_TILE = NP // NS


_sc_mesh = plsc.VectorSubcoreMesh(core_axis_name="c", subcore_axis_name="s")


@functools.partial(
    pl.kernel,
    out_type=jax.ShapeDtypeStruct((NC * NP, H), jnp.float32),
    mesh=_sc_mesh,
    scratch_types=[
        pltpu.VMEM((OPS_PER_TILE, CHUNK), jnp.int32),
        pltpu.VMEM((OPS_PER_TILE, CHUNK), jnp.int32),
        pltpu.VMEM((CHUNK, H), jnp.float32),
        pltpu.VMEM((CHUNK, H), jnp.float32),
        pltpu.VMEM((CHUNK, H), jnp.float32),
        pltpu.VMEM((CHUNK, H), jnp.float32),
        pltpu.VMEM_SHARED((NP, H), jnp.float32),
        pltpu.VMEM_SHARED((NP, H), jnp.float32),
        pltpu.SemaphoreType.DMA,
        pltpu.SemaphoreType.DMA,
        pltpu.SemaphoreType.DMA,
        pltpu.SemaphoreType.DMA,
        pltpu.SemaphoreType.DMA,
        pltpu.SemaphoreType.DMA,
        pltpu.SemaphoreType.DMA,
        pltpu.SemaphoreType.DMA,
    ],
    compiler_params=pltpu.CompilerParams(use_tc_tiling_on_sc=False),
)
def _sc_segment_sum(z_hbm, src_hbm, dst_hbm, zeros_hbm, out_hbm,
                    sidx, didx, rb0, rb1, rb2, rb3, zsp, acc,
                    g0, g1, g2, g3, s0, s1, s2, s3):
    c = lax.axis_index("c")
    s = lax.axis_index("s")
    wid = s * NC + c
    r0 = s * ROWS_PER_TILE

    st_src = pltpu.make_async_copy(
        src_hbm.at[pl.ds(wid * OPS_PER_TILE, OPS_PER_TILE)], sidx, g0)
    st_dst = pltpu.make_async_copy(
        dst_hbm.at[pl.ds(wid * OPS_PER_TILE, OPS_PER_TILE)], didx, g1)
    st_z = pltpu.make_async_copy(z_hbm.at[pl.ds(r0, ROWS_PER_TILE)],
                                 zsp.at[pl.ds(r0, ROWS_PER_TILE)], g2)
    st_acc = pltpu.make_async_copy(zeros_hbm.at[pl.ds(r0, ROWS_PER_TILE)],
                                   acc.at[pl.ds(r0, ROWS_PER_TILE)], g3)
    st_src.start()
    st_dst.start()
    st_z.start()
    st_acc.start()
    st_src.wait()
    st_dst.wait()
    st_z.wait()
    st_acc.wait()
    plsc.subcore_barrier()

    bufs = (rb0, rb1, rb2, rb3)
    gsem = (g0, g1, g2, g3)
    ssem = (s0, s1, s2, s3)

    def _drain(j, sems):
        pltpu.make_async_copy(z_hbm.at[pl.ds(0, CHUNK)], bufs[j],
                              sems[j]).wait()

    def _gat(k, j):
        pltpu.async_copy(zsp.at[sidx.at[k]], bufs[j], gsem[j])

    def _scat(k, j):
        pltpu.async_copy(bufs[j], acc.at[didx.at[k]], ssem[j], add=True)

    def _step(k, j, lookahead):
        _drain(j, gsem)
        _scat(k, j)
        if lookahead:
            _drain((j + 2) % 4, ssem)
            _gat(k + 2, (j + 2) % 4)

    _gat(0, 0)
    _gat(1, 1)
    _drain(0, gsem); _scat(0, 0); _gat(2, 2)
    _drain(1, gsem); _scat(1, 1); _gat(3, 3)
    _step(2, 2, True)
    _step(3, 3, True)

    def body(b, carry):
        k = 4 * b
        _step(k, 0, True)
        _step(k + 1, 1, True)
        _step(k + 2, 2, True)
        _step(k + 3, 3, True)
        return carry

    lax.fori_loop(1, OPS_PER_TILE // 4 - 1, body, 0)

    _step(OPS_PER_TILE - 4, 0, True)
    _step(OPS_PER_TILE - 3, 1, True)
    _drain(2, gsem); _scat(OPS_PER_TILE - 2, 2)
    _drain(3, gsem); _scat(OPS_PER_TILE - 1, 3)
    _drain(0, ssem)
    _drain(1, ssem)
    _drain(2, ssem)
    _drain(3, ssem)
    plsc.subcore_barrier()

    pltpu.sync_copy(acc.at[pl.ds(r0, ROWS_PER_TILE)],
                    out_hbm.at[pl.ds(c * NP + r0, ROWS_PER_TILE)])


NPR = NP // 4
NRR = N // 4
PW = 4 * H


def _fold4(v):
    return v[:, 0:H] + v[:, H:2 * H] + v[:, 2 * H:3 * H] + v[:, 3 * H:4 * H]


def _tile4(v):
    return jnp.concatenate([v, v, v, v], axis=1)


def _proj_body(x_ref, w_ref, o_ref):
    o_ref[...] = jnp.dot(x_ref[...], w_ref[...],
                         preferred_element_type=jnp.float32)


def _proj(x4, w1bd):
    return pl.pallas_call(
        _proj_body,
        out_shape=jax.ShapeDtypeStruct((NPR, PW), jnp.float32),
    )(x4, w1bd)


def _bn_tail(pre, gamma_ref, beta_ref, w2bd_ref, b2_ref):
    mean = _tile4(_fold4(jnp.sum(pre, axis=0, keepdims=True)) / N)
    d = pre - mean
    var = _tile4(_fold4(jnp.sum(d * d, axis=0, keepdims=True)) / N)
    hn = d * lax.rsqrt(var + 1e-5) * gamma_ref[...] + beta_ref[...]
    hn = jnp.maximum(hn, 0.0)
    h2 = jnp.dot(hn, w2bd_ref[...],
                 preferred_element_type=jnp.float32) + b2_ref[...]
    return jnp.maximum(h2, 0.0)


def _mid_body(z_ref, part_ref, b1_ref, gamma_ref, beta_ref, w2bd_ref, b2_ref,
              wnbd_ref, o_ref):
    pre = (z_ref[:NRR, :] + part_ref[:NRR, :]
           + part_ref[NPR:NPR + NRR, :] + b1_ref[...])
    h2 = _bn_tail(pre, gamma_ref, beta_ref, w2bd_ref, b2_ref)
    o_ref[:NRR, :] = jnp.dot(h2, wnbd_ref[...],
                             preferred_element_type=jnp.float32)
    o_ref[NRR:, :] = jnp.zeros((NPR - NRR, PW), jnp.float32)


def _mid(z, partp, pp):
    return pl.pallas_call(
        _mid_body,
        out_shape=jax.ShapeDtypeStruct((NPR, PW), jnp.float32),
    )(z, partp, pp["b1"], pp["gamma"], pp["beta"], pp["W2bd"], pp["b2"],
      pp["Wnbd"])


def _last_body(z_ref, part_ref, b1_ref, gamma_ref, beta_ref, w2bd_ref, b2_ref,
               hw1bd_ref, batch_ref, hb1_ref, hw2_ref, hb2_ref, o_ref):
    pre = (z_ref[:NRR, :] + part_ref[:NRR, :]
           + part_ref[NPR:NPR + NRR, :] + b1_ref[...])
    h2 = _bn_tail(pre, gamma_ref, beta_ref, w2bd_ref, b2_ref)
    hz = jnp.dot(h2, hw1bd_ref[...], preferred_element_type=jnp.float32)
    gids = lax.broadcasted_iota(jnp.int32, (G, NRR), 0)
    pooled = jnp.zeros((G, H), jnp.float32)
    for j in range(4):
        onehot = jnp.where(gids == batch_ref[j:j + 1, :NRR], 1.0, 0.0)
        pooled = pooled + jnp.dot(onehot, hz[:, j * H:(j + 1) * H],
                                  preferred_element_type=jnp.float32)
    hh = jnp.maximum(pooled + hb1_ref[...], 0.0)
    o_ref[...] = jnp.dot(hh, hw2_ref[...],
                         preferred_element_type=jnp.float32) + hb2_ref[...]


def _last(z, partp, pp, head, batchp):
    return pl.pallas_call(
        _last_body,
        out_shape=jax.ShapeDtypeStruct((G, D_OUT), jnp.float32),
    )(z, partp, pp["b1"], pp["gamma"], pp["beta"], pp["W2bd"], pp["b2"],
      jnp.kron(jnp.eye(4, dtype=jnp.float32), head["W1"]), batchp,
      head["b1"].reshape(1, H), head["W2"], head["b2"].reshape(1, D_OUT))


def _pack_params(p, w_next):
    eye4 = jnp.eye(4, dtype=jnp.float32)
    return {
        "b1": _tile4(p["b1"].reshape(1, H)),
        "gamma": _tile4(p["gamma"].reshape(1, H)),
        "beta": _tile4(p["beta"].reshape(1, H)),
        "W2bd": jnp.kron(eye4, p["W2"]),
        "b2": _tile4(p["b2"].reshape(1, H)),
        "Wnbd": None if w_next is None else jnp.kron(eye4, w_next),
    }


def kernel(x, edge_index, batch, params):
    epad = jnp.full((EP - E,), N, jnp.int32)
    src = jnp.concatenate([edge_index[0], epad]).reshape(EP // CHUNK, CHUNK)
    dst = jnp.concatenate([edge_index[1], epad]).reshape(EP // CHUNK, CHUNK)
    convs = params["convs"]
    zeros = jnp.zeros((NP, H), jnp.float32)
    x4 = jnp.pad(x, ((0, NP - N), (0, 0))).reshape(NPR, 4 * D_IN)
    w1bd0 = jnp.kron(jnp.eye(4, dtype=jnp.float32), convs[0]["W1"])
    batchp = jnp.concatenate(
        [batch, jnp.full((NP - N,), G, jnp.int32)]).reshape(NPR, 4).T
    z = _proj(x4, w1bd0)
    for l in range(5):
        part = _sc_segment_sum(z.reshape(NP, H), src, dst, zeros)
        partp = part.reshape(2 * NPR, PW)
        if l < 4:
            z = _mid(z, partp, _pack_params(convs[l], convs[l + 1]["W1"]))
        else:
            out = _last(z, partp, _pack_params(convs[4], None),
                        params["head"], batchp)
    return out


# speedup vs baseline: 27.6902x; 1.0100x over previous
import functools

import jax
import jax.numpy as jnp
from jax import lax
from jax.experimental import pallas as pl
from jax.experimental.pallas import tpu as pltpu
from jax.experimental.pallas import tpu_sc as plsc

N = 10000
E = 320000
D_IN = 128
H = 32
G = 64
D_OUT = 128

NC = 2
NS = 16
NW = NC * NS

CHUNK = 128
EP = 327680
OPS_PER_TILE = EP // (NW * CHUNK)
NP = 10240
ROWS_PER_TILE = NP // NS


_sc_mesh = plsc.VectorSubcoreMesh(core_axis_name="c", subcore_axis_name="s")


@functools.partial(
    pl.kernel,
    out_type=jax.ShapeDtypeStruct((NC * NP, H), jnp.float32),
    mesh=_sc_mesh,
    scratch_types=[
        pltpu.VMEM((OPS_PER_TILE, CHUNK), jnp.int32),
        pltpu.VMEM((OPS_PER_TILE, CHUNK), jnp.int32),
        pltpu.VMEM((CHUNK, H), jnp.float32),
        pltpu.VMEM((CHUNK, H), jnp.float32),
        pltpu.VMEM((CHUNK, H), jnp.float32),
        pltpu.VMEM((CHUNK, H), jnp.float32),
        pltpu.VMEM_SHARED((NP, H), jnp.float32),
        pltpu.VMEM_SHARED((NP, H), jnp.float32),
        pltpu.SemaphoreType.DMA,
        pltpu.SemaphoreType.DMA,
        pltpu.SemaphoreType.DMA,
        pltpu.SemaphoreType.DMA,
        pltpu.SemaphoreType.DMA,
        pltpu.SemaphoreType.DMA,
        pltpu.SemaphoreType.DMA,
        pltpu.SemaphoreType.DMA,
    ],
    compiler_params=pltpu.CompilerParams(use_tc_tiling_on_sc=False),
)
def _sc_segment_sum(z_hbm, src_hbm, dst_hbm, zeros_hbm, out_hbm,
                    sidx, didx, rb0, rb1, rb2, rb3, zsp, acc,
                    g0, g1, g2, g3, s0, s1, s2, s3):
    c = lax.axis_index("c")
    s = lax.axis_index("s")
    wid = s * NC + c
    r0 = s * ROWS_PER_TILE

    st_src = pltpu.make_async_copy(
        src_hbm.at[pl.ds(wid * OPS_PER_TILE, OPS_PER_TILE)], sidx, g0)
    st_dst = pltpu.make_async_copy(
        dst_hbm.at[pl.ds(wid * OPS_PER_TILE, OPS_PER_TILE)], didx, g1)
    st_z = pltpu.make_async_copy(z_hbm.at[pl.ds(r0, ROWS_PER_TILE)],
                                 zsp.at[pl.ds(r0, ROWS_PER_TILE)], g2)
    st_acc = pltpu.make_async_copy(zeros_hbm.at[pl.ds(r0, ROWS_PER_TILE)],
                                   acc.at[pl.ds(r0, ROWS_PER_TILE)], g3)
    st_src.start()
    st_dst.start()
    st_z.start()
    st_acc.start()
    st_src.wait()
    st_dst.wait()
    st_z.wait()
    st_acc.wait()
    plsc.subcore_barrier()

    bufs = (rb0, rb1, rb2, rb3)
    gsem = (g0, g1, g2, g3)
    ssem = (s0, s1, s2, s3)

    def _drain(j, sems):
        pltpu.make_async_copy(z_hbm.at[pl.ds(0, CHUNK)], bufs[j],
                              sems[j]).wait()

    def _gat(k, j):
        pltpu.async_copy(zsp.at[sidx.at[k]], bufs[j], gsem[j])

    def _scat(k, j):
        pltpu.async_copy(bufs[j], acc.at[didx.at[k]], ssem[j], add=True)

    def _step(k, j, lookahead):
        _drain(j, gsem)
        _scat(k, j)
        if lookahead:
            _drain((j + 2) % 4, ssem)
            _gat(k + 2, (j + 2) % 4)

    _gat(0, 0)
    _gat(1, 1)
    _drain(0, gsem); _scat(0, 0); _gat(2, 2)
    _drain(1, gsem); _scat(1, 1); _gat(3, 3)
    _step(2, 2, True)
    _step(3, 3, True)

    def body(b, carry):
        k = 4 * b
        _step(k, 0, True)
        _step(k + 1, 1, True)
        _step(k + 2, 2, True)
        _step(k + 3, 3, True)
        return carry

    lax.fori_loop(1, OPS_PER_TILE // 4 - 1, body, 0)

    _step(OPS_PER_TILE - 4, 0, True)
    _step(OPS_PER_TILE - 3, 1, True)
    _drain(2, gsem); _scat(OPS_PER_TILE - 2, 2)
    _drain(3, gsem); _scat(OPS_PER_TILE - 1, 3)
    _drain(0, ssem)
    _drain(1, ssem)
    _drain(2, ssem)
    _drain(3, ssem)
    plsc.subcore_barrier()

    pltpu.sync_copy(acc.at[pl.ds(r0, ROWS_PER_TILE)],
                    out_hbm.at[pl.ds(c * NP + r0, ROWS_PER_TILE)])


NPR = NP // 4
NRR = N // 4
PW = 4 * H


def _fold4(v):
    return v[:, 0:H] + v[:, H:2 * H] + v[:, 2 * H:3 * H] + v[:, 3 * H:4 * H]


def _tile4(v):
    return jnp.concatenate([v, v, v, v], axis=1)


def _proj_body(x_ref, w_ref, o_ref):
    o_ref[...] = jnp.dot(x_ref[...], w_ref[...],
                         preferred_element_type=jnp.float32)


def _proj(x4, w1bd):
    return pl.pallas_call(
        _proj_body,
        out_shape=jax.ShapeDtypeStruct((NPR, PW), jnp.float32),
    )(x4, w1bd)


def _bn_tail(pre, gamma_ref, beta_ref, w2bd_ref, b2_ref):
    s1 = _fold4(jnp.sum(pre, axis=0, keepdims=True)) / N
    s2 = _fold4(jnp.sum(pre * pre, axis=0, keepdims=True)) / N
    mean = _tile4(s1)
    var = _tile4(s2 - s1 * s1)
    hn = (pre - mean) * lax.rsqrt(var + 1e-5) * gamma_ref[...] + beta_ref[...]
    hn = jnp.maximum(hn, 0.0)
    h2 = jnp.dot(hn, w2bd_ref[...],
                 preferred_element_type=jnp.float32) + b2_ref[...]
    return jnp.maximum(h2, 0.0)


def _mid_body(z_ref, part_ref, b1_ref, gamma_ref, beta_ref, w2bd_ref, b2_ref,
              wnbd_ref, o_ref):
    pre = (z_ref[:NRR, :] + part_ref[:NRR, :]
           + part_ref[NPR:NPR + NRR, :] + b1_ref[...])
    h2 = _bn_tail(pre, gamma_ref, beta_ref, w2bd_ref, b2_ref)
    o_ref[:NRR, :] = jnp.dot(h2, wnbd_ref[...],
                             preferred_element_type=jnp.float32)
    o_ref[NRR:, :] = jnp.zeros((NPR - NRR, PW), jnp.float32)


def _mid(z, partp, pp):
    return pl.pallas_call(
        _mid_body,
        out_shape=jax.ShapeDtypeStruct((NPR, PW), jnp.float32),
    )(z, partp, pp["b1"], pp["gamma"], pp["beta"], pp["W2bd"], pp["b2"],
      pp["Wnbd"])


def _last_body(z_ref, part_ref, b1_ref, gamma_ref, beta_ref, w2bd_ref, b2_ref,
               hw1bd_ref, batch_ref, hb1_ref, hw2_ref, hb2_ref, o_ref):
    pre = (z_ref[:NRR, :] + part_ref[:NRR, :]
           + part_ref[NPR:NPR + NRR, :] + b1_ref[...])
    h2 = _bn_tail(pre, gamma_ref, beta_ref, w2bd_ref, b2_ref)
    hz = jnp.dot(h2, hw1bd_ref[...], preferred_element_type=jnp.float32)
    gids = lax.broadcasted_iota(jnp.int32, (G, NRR), 0)
    pooled = jnp.zeros((G, H), jnp.float32)
    for j in range(4):
        onehot = jnp.where(gids == batch_ref[j:j + 1, :NRR], 1.0, 0.0)
        pooled = pooled + jnp.dot(onehot, hz[:, j * H:(j + 1) * H],
                                  preferred_element_type=jnp.float32)
    hh = jnp.maximum(pooled + hb1_ref[...], 0.0)
    o_ref[...] = jnp.dot(hh, hw2_ref[...],
                         preferred_element_type=jnp.float32) + hb2_ref[...]


def _last(z, partp, pp, head, batchp):
    return pl.pallas_call(
        _last_body,
        out_shape=jax.ShapeDtypeStruct((G, D_OUT), jnp.float32),
    )(z, partp, pp["b1"], pp["gamma"], pp["beta"], pp["W2bd"], pp["b2"],
      jnp.kron(jnp.eye(4, dtype=jnp.float32), head["W1"]), batchp,
      head["b1"].reshape(1, H), head["W2"], head["b2"].reshape(1, D_OUT))


def _pack_params(p, w_next):
    eye4 = jnp.eye(4, dtype=jnp.float32)
    return {
        "b1": _tile4(p["b1"].reshape(1, H)),
        "gamma": _tile4(p["gamma"].reshape(1, H)),
        "beta": _tile4(p["beta"].reshape(1, H)),
        "W2bd": jnp.kron(eye4, p["W2"]),
        "b2": _tile4(p["b2"].reshape(1, H)),
        "Wnbd": None if w_next is None else jnp.kron(eye4, w_next),
    }


def kernel(x, edge_index, batch, params):
    epad = jnp.full((EP - E,), N, jnp.int32)
    src = jnp.concatenate([edge_index[0], epad]).reshape(EP // CHUNK, CHUNK)
    dst = jnp.concatenate([edge_index[1], epad]).reshape(EP // CHUNK, CHUNK)
    convs = params["convs"]
    zeros = jnp.zeros((NP, H), jnp.float32)
    x4 = jnp.pad(x, ((0, NP - N), (0, 0))).reshape(NPR, 4 * D_IN)
    w1bd0 = jnp.kron(jnp.eye(4, dtype=jnp.float32), convs[0]["W1"])
    batchp = jnp.concatenate(
        [batch, jnp.full((NP - N,), G, jnp.int32)]).reshape(NPR, 4).T
    z = _proj(x4, w1bd0)
    for l in range(5):
        part = _sc_segment_sum(z.reshape(NP, H), src, dst, zeros)
        partp = part.reshape(2 * NPR, PW)
        if l < 4:
            z = _mid(z, partp, _pack_params(convs[l], convs[l + 1]["W1"]))
        else:
            out = _last(z, partp, _pack_params(convs[4], None),
                        params["head"], batchp)
    return out


# speedup vs baseline: 27.8678x; 1.0064x over previous
import functools

import jax
import jax.numpy as jnp
from jax import lax
from jax.experimental import pallas as pl
from jax.experimental.pallas import tpu as pltpu
from jax.experimental.pallas import tpu_sc as plsc

N = 10000
E = 320000
D_IN = 128
H = 32
G = 64
D_OUT = 128

NC = 2
NS = 16
NW = NC * NS

CHUNK = 128
OPS_PER_TILE = 80
REAL_CHUNKS = E // CHUNK
LAST_REAL = REAL_CHUNKS - (NW - 1) * OPS_PER_TILE
PAD_OPS = OPS_PER_TILE - LAST_REAL
NP = 10240
ROWS_PER_TILE = NP // NS


_sc_mesh = plsc.VectorSubcoreMesh(core_axis_name="c", subcore_axis_name="s")


@functools.partial(
    pl.kernel,
    out_type=jax.ShapeDtypeStruct((NC * NP, H), jnp.float32),
    mesh=_sc_mesh,
    scratch_types=[
        pltpu.VMEM((OPS_PER_TILE, CHUNK), jnp.int32),
        pltpu.VMEM((OPS_PER_TILE, CHUNK), jnp.int32),
        pltpu.VMEM((CHUNK, H), jnp.float32),
        pltpu.VMEM((CHUNK, H), jnp.float32),
        pltpu.VMEM((CHUNK, H), jnp.float32),
        pltpu.VMEM((CHUNK, H), jnp.float32),
        pltpu.VMEM_SHARED((NP, H), jnp.float32),
        pltpu.VMEM_SHARED((NP, H), jnp.float32),
        pltpu.SemaphoreType.DMA,
        pltpu.SemaphoreType.DMA,
        pltpu.SemaphoreType.DMA,
        pltpu.SemaphoreType.DMA,
        pltpu.SemaphoreType.DMA,
        pltpu.SemaphoreType.DMA,
        pltpu.SemaphoreType.DMA,
        pltpu.SemaphoreType.DMA,
    ],
    compiler_params=pltpu.CompilerParams(use_tc_tiling_on_sc=False),
)
def _sc_segment_sum(z_hbm, src_hbm, dst_hbm, zeros_hbm, pad_hbm, out_hbm,
                    sidx, didx, rb0, rb1, rb2, rb3, zsp, acc,
                    g0, g1, g2, g3, s0, s1, s2, s3):
    c = lax.axis_index("c")
    s = lax.axis_index("s")
    wid = s * NC + c
    r0 = s * ROWS_PER_TILE

    st_z = pltpu.make_async_copy(z_hbm.at[pl.ds(r0, ROWS_PER_TILE)],
                                 zsp.at[pl.ds(r0, ROWS_PER_TILE)], g2)
    st_acc = pltpu.make_async_copy(zeros_hbm.at[pl.ds(r0, ROWS_PER_TILE)],
                                   acc.at[pl.ds(r0, ROWS_PER_TILE)], g3)
    st_z.start()
    st_acc.start()

    @pl.when(wid < NW - 1)
    def _():
        a = pltpu.make_async_copy(
            src_hbm.at[pl.ds(wid * OPS_PER_TILE, OPS_PER_TILE)], sidx, g0)
        b = pltpu.make_async_copy(
            dst_hbm.at[pl.ds(wid * OPS_PER_TILE, OPS_PER_TILE)], didx, g1)
        a.start()
        b.start()
        a.wait()
        b.wait()

    @pl.when(wid == NW - 1)
    def _():
        a = pltpu.make_async_copy(
            src_hbm.at[pl.ds((NW - 1) * OPS_PER_TILE, LAST_REAL)],
            sidx.at[pl.ds(0, LAST_REAL)], g0)
        b = pltpu.make_async_copy(
            dst_hbm.at[pl.ds((NW - 1) * OPS_PER_TILE, LAST_REAL)],
            didx.at[pl.ds(0, LAST_REAL)], g1)
        cpad = pltpu.make_async_copy(
            pad_hbm, sidx.at[pl.ds(LAST_REAL, PAD_OPS)], s0)
        dpad = pltpu.make_async_copy(
            pad_hbm, didx.at[pl.ds(LAST_REAL, PAD_OPS)], s1)
        a.start()
        b.start()
        cpad.start()
        dpad.start()
        a.wait()
        b.wait()
        cpad.wait()
        dpad.wait()

    st_z.wait()
    st_acc.wait()
    plsc.subcore_barrier()

    bufs = (rb0, rb1, rb2, rb3)
    gsem = (g0, g1, g2, g3)
    ssem = (s0, s1, s2, s3)

    def _drain(j, sems):
        pltpu.make_async_copy(z_hbm.at[pl.ds(0, CHUNK)], bufs[j],
                              sems[j]).wait()

    def _gat(k, j):
        pltpu.async_copy(zsp.at[sidx.at[k]], bufs[j], gsem[j])

    def _scat(k, j):
        pltpu.async_copy(bufs[j], acc.at[didx.at[k]], ssem[j], add=True)

    def _step(k, j, lookahead):
        _drain(j, gsem)
        _scat(k, j)
        if lookahead:
            _drain((j + 2) % 4, ssem)
            _gat(k + 2, (j + 2) % 4)

    _gat(0, 0)
    _gat(1, 1)
    _drain(0, gsem); _scat(0, 0); _gat(2, 2)
    _drain(1, gsem); _scat(1, 1); _gat(3, 3)
    _step(2, 2, True)
    _step(3, 3, True)

    def body(b, carry):
        k = 4 * b
        _step(k, 0, True)
        _step(k + 1, 1, True)
        _step(k + 2, 2, True)
        _step(k + 3, 3, True)
        return carry

    lax.fori_loop(1, OPS_PER_TILE // 4 - 1, body, 0)

    _step(OPS_PER_TILE - 4, 0, True)
    _step(OPS_PER_TILE - 3, 1, True)
    _drain(2, gsem); _scat(OPS_PER_TILE - 2, 2)
    _drain(3, gsem); _scat(OPS_PER_TILE - 1, 3)
    _drain(0, ssem)
    _drain(1, ssem)
    _drain(2, ssem)
    _drain(3, ssem)
    plsc.subcore_barrier()

    pltpu.sync_copy(acc.at[pl.ds(r0, ROWS_PER_TILE)],
                    out_hbm.at[pl.ds(c * NP + r0, ROWS_PER_TILE)])


NPR = NP // 4
NRR = N // 4
PW = 4 * H


def _fold4(v):
    return v[:, 0:H] + v[:, H:2 * H] + v[:, 2 * H:3 * H] + v[:, 3 * H:4 * H]


def _tile4(v):
    return jnp.concatenate([v, v, v, v], axis=1)


def _proj_body(x_ref, w_ref, o_ref):
    o_ref[:NRR, :] = jnp.dot(x_ref[...], w_ref[...],
                             preferred_element_type=jnp.float32)
    o_ref[NRR:, :] = jnp.zeros((NPR - NRR, PW), jnp.float32)


def _proj(x4, w1bd):
    return pl.pallas_call(
        _proj_body,
        out_shape=jax.ShapeDtypeStruct((NPR, PW), jnp.float32),
    )(x4, w1bd)


def _bn_tail(pre, gamma_ref, beta_ref, w2bd_ref, b2_ref):
    s1 = _fold4(jnp.sum(pre, axis=0, keepdims=True)) / N
    s2 = _fold4(jnp.sum(pre * pre, axis=0, keepdims=True)) / N
    mean = _tile4(s1)
    var = _tile4(s2 - s1 * s1)
    hn = (pre - mean) * lax.rsqrt(var + 1e-5) * gamma_ref[...] + beta_ref[...]
    hn = jnp.maximum(hn, 0.0)
    h2 = jnp.dot(hn, w2bd_ref[...],
                 preferred_element_type=jnp.float32) + b2_ref[...]
    return jnp.maximum(h2, 0.0)


def _mid_body(z_ref, part_ref, b1_ref, gamma_ref, beta_ref, w2bd_ref, b2_ref,
              wnbd_ref, o_ref):
    pre = (z_ref[:NRR, :] + part_ref[:NRR, :]
           + part_ref[NPR:NPR + NRR, :] + b1_ref[...])
    h2 = _bn_tail(pre, gamma_ref, beta_ref, w2bd_ref, b2_ref)
    o_ref[:NRR, :] = jnp.dot(h2, wnbd_ref[...],
                             preferred_element_type=jnp.float32)
    o_ref[NRR:, :] = jnp.zeros((NPR - NRR, PW), jnp.float32)


def _mid(z, partp, pp):
    return pl.pallas_call(
        _mid_body,
        out_shape=jax.ShapeDtypeStruct((NPR, PW), jnp.float32),
    )(z, partp, pp["b1"], pp["gamma"], pp["beta"], pp["W2bd"], pp["b2"],
      pp["Wnbd"])


def _last_body(z_ref, part_ref, b1_ref, gamma_ref, beta_ref, w2bd_ref, b2_ref,
               hw1bd_ref, batch_ref, hb1_ref, hw2_ref, hb2_ref, o_ref):
    pre = (z_ref[:NRR, :] + part_ref[:NRR, :]
           + part_ref[NPR:NPR + NRR, :] + b1_ref[...])
    h2 = _bn_tail(pre, gamma_ref, beta_ref, w2bd_ref, b2_ref)
    hz = jnp.dot(h2, hw1bd_ref[...], preferred_element_type=jnp.float32)
    gids = lax.broadcasted_iota(jnp.int32, (G, NRR), 0)
    pooled = jnp.zeros((G, H), jnp.float32)
    for j in range(4):
        onehot = jnp.where(gids == batch_ref[j:j + 1, :NRR], 1.0, 0.0)
        pooled = pooled + jnp.dot(onehot, hz[:, j * H:(j + 1) * H],
                                  preferred_element_type=jnp.float32)
    hh = jnp.maximum(pooled + hb1_ref[...], 0.0)
    o_ref[...] = jnp.dot(hh, hw2_ref[...],
                         preferred_element_type=jnp.float32) + hb2_ref[...]


def _last(z, partp, pp, head, batchp):
    return pl.pallas_call(
        _last_body,
        out_shape=jax.ShapeDtypeStruct((G, D_OUT), jnp.float32),
    )(z, partp, pp["b1"], pp["gamma"], pp["beta"], pp["W2bd"], pp["b2"],
      jnp.kron(jnp.eye(4, dtype=jnp.float32), head["W1"]), batchp,
      head["b1"].reshape(1, H), head["W2"], head["b2"].reshape(1, D_OUT))


def _pack_params(p, w_next):
    eye4 = jnp.eye(4, dtype=jnp.float32)
    return {
        "b1": _tile4(p["b1"].reshape(1, H)),
        "gamma": _tile4(p["gamma"].reshape(1, H)),
        "beta": _tile4(p["beta"].reshape(1, H)),
        "W2bd": jnp.kron(eye4, p["W2"]),
        "b2": _tile4(p["b2"].reshape(1, H)),
        "Wnbd": None if w_next is None else jnp.kron(eye4, w_next),
    }


def kernel(x, edge_index, batch, params):
    src = edge_index[0].reshape(REAL_CHUNKS, CHUNK)
    dst = edge_index[1].reshape(REAL_CHUNKS, CHUNK)
    convs = params["convs"]
    zeros = jnp.zeros((NP, H), jnp.float32)
    pad_idx = jnp.full((PAD_OPS, CHUNK), N, jnp.int32)
    w1bd0 = jnp.kron(jnp.eye(4, dtype=jnp.float32), convs[0]["W1"])
    batchp = jnp.concatenate(
        [batch, jnp.full((NP - N,), G, jnp.int32)]).reshape(NPR, 4).T
    z = _proj(x.reshape(NRR, 4 * D_IN), w1bd0)
    for l in range(5):
        part = _sc_segment_sum(z.reshape(NP, H), src, dst, zeros, pad_idx)
        partp = part.reshape(2 * NPR, PW)
        if l < 4:
            z = _mid(z, partp, _pack_params(convs[l], convs[l + 1]["W1"]))
        else:
            out = _last(z, partp, _pack_params(convs[4], None),
                        params["head"], batchp)
    return out
